# baseline (speedup 1.0000x reference)
import functools

import jax
import jax.numpy as jnp
from jax import lax
from jax.experimental import pallas as pl
from jax.experimental.pallas import tpu as pltpu
from jax.experimental.pallas import tpu_sc as plsc

VOCAB = 1000000
EMBED = 64
BATCH = 16384
LANES = 16

try:
    _info = plsc.get_sparse_core_info()
    _NC, _NS = _info.num_cores, _info.num_subcores
except Exception:
    _NC, _NS = 2, 16
_NW = _NC * _NS
_BPW = BATCH // _NW


def _build_sc_kernel():
    mesh = plsc.VectorSubcoreMesh(core_axis_name="c", subcore_axis_name="s")

    @functools.partial(
        pl.kernel,
        mesh=mesh,
        out_type=jax.ShapeDtypeStruct((BATCH,), jnp.float32),
        scratch_types=[
            pltpu.VMEM((_BPW,), jnp.int32),
            pltpu.VMEM((_BPW,), jnp.int32),
            pltpu.VMEM((EMBED * LANES,), jnp.int32),
            pltpu.VMEM((EMBED * LANES,), jnp.int32),
            pltpu.VMEM((EMBED * LANES,), jnp.float32),
            pltpu.VMEM((EMBED * LANES,), jnp.float32),
            pltpu.VMEM((_BPW,), jnp.float32),
            pltpu.SemaphoreType.DMA,
        ],
        compiler_params=pltpu.CompilerParams(use_tc_tiling_on_sc=False),
    )
    def sc_kernel(center_hbm, context_hbm, ctab_hbm, xtab_hbm, out_hbm,
                  cidx_v, xidx_v, cil_v, xil_v, cbuf_v, xbuf_v, score_v, sem):
        wid = lax.axis_index("s") * _NC + lax.axis_index("c")
        base = wid * _BPW
        ngrp = _BPW // LANES
        nchunk = EMBED * LANES // 128

        pltpu.sync_copy(center_hbm.at[pl.ds(base, _BPW)], cidx_v)
        pltpu.sync_copy(context_hbm.at[pl.ds(base, _BPW)], xidx_v)

        def group_body(g, carry):
            b0 = g * LANES
            cvec = cidx_v[pl.ds(b0, LANES)]
            uvec = xidx_v[pl.ds(b0, LANES)]
            for d in range(EMBED):
                off = jnp.full((LANES,), d * VOCAB, jnp.int32)
                cil_v[pl.ds(d * LANES, LANES)] = off + cvec
                xil_v[pl.ds(d * LANES, LANES)] = off + uvec
            copies = []
            for k in range(nchunk):
                sl = pl.ds(k * 128, 128)
                copies.append(pltpu.async_copy(
                    ctab_hbm.at[cil_v.at[sl]], cbuf_v.at[sl], sem))
                copies.append(pltpu.async_copy(
                    xtab_hbm.at[xil_v.at[sl]], xbuf_v.at[sl], sem))
            for cp in copies:
                cp.wait()
            acc = (cbuf_v[pl.ds(0, LANES)] * xbuf_v[pl.ds(0, LANES)])
            for d in range(1, EMBED):
                acc = acc + (cbuf_v[pl.ds(d * LANES, LANES)]
                             * xbuf_v[pl.ds(d * LANES, LANES)])
            score_v[pl.ds(b0, LANES)] = acc
            return carry

        lax.fori_loop(0, ngrp, group_body, 0)

        pltpu.sync_copy(score_v, out_hbm.at[pl.ds(base, _BPW)])

    return sc_kernel


_sc_kernel = _build_sc_kernel()


def kernel(center, context, center_table, context_table):
    return _sc_kernel(center.astype(jnp.int32), context.astype(jnp.int32),
                      center_table.T.reshape(-1), context_table.T.reshape(-1))


# speedup vs baseline: 3.1412x; 3.1412x over previous
import functools

import jax
import jax.numpy as jnp
from jax import lax
from jax.experimental import pallas as pl
from jax.experimental.pallas import tpu as pltpu
from jax.experimental.pallas import tpu_sc as plsc

VOCAB = 1000000
EMBED = 64
BATCH = 16384
LANES = 16
PACKED = EMBED // 2
IDX_CHUNK = 128

try:
    _info = plsc.get_sparse_core_info()
    _NC, _NS = _info.num_cores, _info.num_subcores
except Exception:
    _NC, _NS = 2, 16
_NW = _NC * _NS
_BPW = BATCH // _NW

_mesh = plsc.VectorSubcoreMesh(core_axis_name="c", subcore_axis_name="s")


def _build_gather_kernel():
    @functools.partial(
        pl.kernel,
        mesh=_mesh,
        out_type=jax.ShapeDtypeStruct((BATCH, PACKED), jnp.int32),
        scratch_types=[
            pltpu.VMEM((_BPW,), jnp.int32),
            pltpu.VMEM((_BPW, PACKED), jnp.int32),
            pltpu.SemaphoreType.DMA,
        ],
        compiler_params=pltpu.CompilerParams(use_tc_tiling_on_sc=False),
    )
    def gather_kernel(idx_hbm, tab_hbm, out_hbm, idx_v, rows_v, sem):
        wid = lax.axis_index("s") * _NC + lax.axis_index("c")
        base = wid * _BPW
        pltpu.sync_copy(idx_hbm.at[pl.ds(base, _BPW)], idx_v)
        copies = []
        for k in range(_BPW // IDX_CHUNK):
            sl = pl.ds(k * IDX_CHUNK, IDX_CHUNK)
            copies.append(pltpu.async_copy(
                tab_hbm.at[idx_v.at[sl]], rows_v.at[sl], sem))
        for cp in copies:
            cp.wait()
        pltpu.sync_copy(rows_v, out_hbm.at[pl.ds(base, _BPW)])

    return gather_kernel


def _build_dot_kernel():
    @functools.partial(
        pl.kernel,
        mesh=_mesh,
        out_type=jax.ShapeDtypeStruct((BATCH,), jnp.float32),
        scratch_types=[
            pltpu.VMEM((_BPW, PACKED), jnp.int32),
            pltpu.VMEM((_BPW, PACKED), jnp.int32),
            pltpu.VMEM((_BPW,), jnp.float32),
        ],
        compiler_params=pltpu.CompilerParams(use_tc_tiling_on_sc=False),
    )
    def dot_kernel(crows_hbm, xrows_hbm, out_hbm, crows_v, xrows_v, score_v):
        wid = lax.axis_index("s") * _NC + lax.axis_index("c")
        base = wid * _BPW
        pltpu.sync_copy(crows_hbm.at[pl.ds(base, _BPW)], crows_v)
        pltpu.sync_copy(xrows_hbm.at[pl.ds(base, _BPW)], xrows_v)

        lane = lax.iota(jnp.int32, LANES)
        dnums = lax.GatherDimensionNumbers(
            offset_dims=(), collapsed_slice_dims=(0,), start_index_map=(0,))
        himask = jnp.full((LANES,), -65536, jnp.int32)

        def unpack(w):
            lo = lax.bitcast_convert_type(
                jnp.left_shift(w, 16), jnp.float32)
            hi = lax.bitcast_convert_type(
                jnp.bitwise_and(w, himask), jnp.float32)
            return lo, hi

        def hsum(vec):
            for s in (1, 2, 4, 8):
                perm = lane ^ s
                vec = vec + lax.gather(
                    vec, perm[:, None], dnums, (1,),
                    mode=lax.GatherScatterMode.PROMISE_IN_BOUNDS)
            return vec

        def group_body(g, carry):
            scores = jnp.zeros((LANES,), jnp.float32)
            for r16 in range(LANES):
                r = g * LANES + r16
                acc = jnp.zeros((LANES,), jnp.float32)
                for j in range(PACKED // LANES):
                    cw = crows_v[r, pl.ds(j * LANES, LANES)]
                    xw = xrows_v[r, pl.ds(j * LANES, LANES)]
                    clo, chi = unpack(cw)
                    xlo, xhi = unpack(xw)
                    acc = acc + clo * xlo + chi * xhi
                scores = jnp.where(lane == r16, hsum(acc), scores)
            score_v[pl.ds(g * LANES, LANES)] = scores
            return carry

        lax.fori_loop(0, _BPW // LANES, group_body, 0)
        pltpu.sync_copy(score_v, out_hbm.at[pl.ds(base, _BPW)])

    return dot_kernel


_gather_kernel = _build_gather_kernel()
_dot_kernel = _build_dot_kernel()


def _pack_table(table):
    packed = table.astype(jnp.bfloat16).reshape(VOCAB, PACKED, 2)
    return lax.bitcast_convert_type(packed, jnp.int32)


def kernel(center, context, center_table, context_table):
    crows = _gather_kernel(center.astype(jnp.int32),
                           _pack_table(center_table))
    xrows = _gather_kernel(context.astype(jnp.int32),
                           _pack_table(context_table))
    return _dot_kernel(crows, xrows)


# speedup vs baseline: 9.1283x; 2.9060x over previous
import functools

import jax
import jax.numpy as jnp
from jax import lax
from jax.experimental import pallas as pl
from jax.experimental.pallas import tpu as pltpu
from jax.experimental.pallas import tpu_sc as plsc

VOCAB = 1000000
EMBED = 64
BATCH = 16384
LANES = 16
PACKED = EMBED // 2
IDX_CHUNK = 128

try:
    _info = plsc.get_sparse_core_info()
    _NC, _NS = _info.num_cores, _info.num_subcores
except Exception:
    _NC, _NS = 2, 16
_NW = _NC * _NS
_BPW = BATCH // _NW

_mesh = plsc.VectorSubcoreMesh(core_axis_name="c", subcore_axis_name="s")


def _build_gather_kernel():
    @functools.partial(
        pl.kernel,
        mesh=_mesh,
        out_type=jax.ShapeDtypeStruct((BATCH, EMBED), jnp.float32),
        scratch_types=[
            pltpu.VMEM((_BPW,), jnp.int32),
            pltpu.VMEM((_BPW, EMBED), jnp.float32),
            pltpu.SemaphoreType.DMA,
        ],
        compiler_params=pltpu.CompilerParams(use_tc_tiling_on_sc=False),
    )
    def gather_kernel(idx_hbm, tab_hbm, out_hbm, idx_v, rows_v, sem):
        wid = lax.axis_index("s") * _NC + lax.axis_index("c")
        base = wid * _BPW
        pltpu.sync_copy(idx_hbm.at[pl.ds(base, _BPW)], idx_v)
        copies = []
        for k in range(_BPW // IDX_CHUNK):
            sl = pl.ds(k * IDX_CHUNK, IDX_CHUNK)
            copies.append(pltpu.async_copy(
                tab_hbm.at[idx_v.at[sl]], rows_v.at[sl], sem))
        for cp in copies:
            cp.wait()
        pltpu.sync_copy(rows_v, out_hbm.at[pl.ds(base, _BPW)])

    return gather_kernel


def _build_dot_kernel():
    @functools.partial(
        pl.kernel,
        mesh=_mesh,
        out_type=jax.ShapeDtypeStruct((BATCH,), jnp.float32),
        scratch_types=[
            pltpu.VMEM((_BPW, EMBED), jnp.float32),
            pltpu.VMEM((_BPW, EMBED), jnp.float32),
            pltpu.VMEM((_BPW,), jnp.float32),
        ],
        compiler_params=pltpu.CompilerParams(use_tc_tiling_on_sc=False),
    )
    def dot_kernel(crows_hbm, xrows_hbm, out_hbm, crows_v, xrows_v, score_v):
        wid = lax.axis_index("s") * _NC + lax.axis_index("c")
        base = wid * _BPW
        pltpu.sync_copy(crows_hbm.at[pl.ds(base, _BPW)], crows_v)
        pltpu.sync_copy(xrows_hbm.at[pl.ds(base, _BPW)], xrows_v)

        lane = lax.iota(jnp.int32, LANES)
        dnums = lax.GatherDimensionNumbers(
            offset_dims=(), collapsed_slice_dims=(0,), start_index_map=(0,))
        himask = jnp.full((LANES,), -65536, jnp.int32)

        def unpack(w):
            lo = lax.bitcast_convert_type(
                jnp.left_shift(w, 16), jnp.float32)
            hi = lax.bitcast_convert_type(
                jnp.bitwise_and(w, himask), jnp.float32)
            return lo, hi

        def hsum(vec):
            for s in (1, 2, 4, 8):
                perm = lane ^ s
                vec = vec + lax.gather(
                    vec, perm[:, None], dnums, (1,),
                    mode=lax.GatherScatterMode.PROMISE_IN_BOUNDS)
            return vec

        def group_body(g, carry):
            scores = jnp.zeros((LANES,), jnp.float32)
            for r16 in range(LANES):
                r = g * LANES + r16
                acc = jnp.zeros((LANES,), jnp.float32)
                for j in range(EMBED // LANES):
                    acc = acc + (crows_v[r, pl.ds(j * LANES, LANES)]
                                 * xrows_v[r, pl.ds(j * LANES, LANES)])
                scores = jnp.where(lane == r16, hsum(acc), scores)
            score_v[pl.ds(g * LANES, LANES)] = scores
            return carry

        lax.fori_loop(0, _BPW // LANES, group_body, 0)
        pltpu.sync_copy(score_v, out_hbm.at[pl.ds(base, _BPW)])

    return dot_kernel


_gather_kernel = _build_gather_kernel()
_dot_kernel = _build_dot_kernel()


def kernel(center, context, center_table, context_table):
    crows = _gather_kernel(center.astype(jnp.int32), center_table)
    xrows = _gather_kernel(context.astype(jnp.int32), context_table)
    return _dot_kernel(crows, xrows)
